# baseline (speedup 1.0000x reference)
import functools

import jax
import jax.numpy as jnp
from jax import lax
from jax.experimental import pallas as pl
from jax.experimental.pallas import tpu as pltpu
from jax.experimental.pallas import tpu_sc as plsc

N = 10000
E = 320000
CH = 128
OUT_CH = 64

NC = 2
NS = 16
NW = NC * NS
CHUNK = 128
G = -(-E // (NW * CHUNK))
EPW = G * CHUNK
E_PAD = EPW * NW
NCHUNKS = NW * G
G_SLOW = 47
G_FAST = 2 * G - G_SLOW
RPS = 640
NROWS = RPS * NS

_sc_mesh = plsc.VectorSubcoreMesh(core_axis_name="c", subcore_axis_name="s")


def _sc_degree(edge_chunks, ones_h, zeros_h):
    @functools.partial(
        pl.kernel,
        out_type=jax.ShapeDtypeStruct((NC, NROWS, 16), jnp.float32),
        mesh=_sc_mesh,
        scratch_types=[
            pltpu.VMEM((2, CHUNK), jnp.int32),
            pltpu.VMEM((CHUNK, 16), jnp.float32),
            pltpu.VMEM_SHARED((NROWS, 16), jnp.float32),
        ],
    )
    def deg_kernel(edges_hbm, ones_hbm, zeros_hbm, out_hbm, idx_v, ones_v,
                   acc):
        c = lax.axis_index("c")
        s = lax.axis_index("s")
        wid = c * NS + s
        pltpu.sync_copy(ones_hbm, ones_v)
        pltpu.sync_copy(zeros_hbm, acc.at[pl.ds(s * RPS, RPS)])
        plsc.subcore_barrier()

        @pl.loop(0, G)
        def _(g):
            pltpu.sync_copy(edges_hbm.at[wid * G + g], idx_v)
            pltpu.sync_copy(ones_v, acc.at[idx_v.at[1]], add=True)

        plsc.subcore_barrier()
        pltpu.sync_copy(
            acc.at[pl.ds(s * RPS, RPS)],
            out_hbm.at[c, pl.ds(s * RPS, RPS)],
        )

    return deg_kernel(edge_chunks, ones_h, zeros_h)


def _sc_spmm(y, edge_chunks, zeros_s):
    @functools.partial(
        pl.kernel,
        out_type=jax.ShapeDtypeStruct((NC, NROWS, CH), jnp.float32),
        mesh=_sc_mesh,
        scratch_types=[
            pltpu.VMEM((2, CHUNK), jnp.int32),
            pltpu.VMEM((2, CHUNK), jnp.int32),
            pltpu.VMEM((CHUNK, CH), jnp.float32),
            pltpu.VMEM((CHUNK, CH), jnp.float32),
            pltpu.VMEM_SHARED((NROWS, CH), jnp.float32),
            pltpu.SemaphoreType.DMA,
            pltpu.SemaphoreType.DMA,
        ],
    )
    def spmm_kernel(y_hbm, edges_hbm, zeros_hbm, out_hbm,
                    idx_a, idx_b, rows_a, rows_b, acc, sem_a, sem_b):
        c = lax.axis_index("c")
        s = lax.axis_index("s")

        pltpu.sync_copy(zeros_hbm, acc.at[pl.ds(s * RPS, RPS)])
        plsc.subcore_barrier()

        def run_range(base0, cnt):
            pltpu.sync_copy(edges_hbm.at[base0], idx_a)
            pltpu.async_copy(y_hbm.at[idx_a.at[0]], rows_a, sem_a)

            @pl.loop(0, cnt - 2, step=2)
            def _(g):
                base = base0 + g
                pltpu.sync_copy(edges_hbm.at[base + 1], idx_b)
                pltpu.async_copy(y_hbm.at[idx_b.at[0]], rows_b, sem_b)
                pltpu.make_async_copy(y_hbm.at[idx_a.at[0]], rows_a,
                                      sem_a).wait()
                pltpu.sync_copy(rows_a, acc.at[idx_a.at[1]], add=True)
                pltpu.sync_copy(edges_hbm.at[base + 2], idx_a)
                pltpu.async_copy(y_hbm.at[idx_a.at[0]], rows_a, sem_a)
                pltpu.make_async_copy(y_hbm.at[idx_b.at[0]], rows_b,
                                      sem_b).wait()
                pltpu.sync_copy(rows_b, acc.at[idx_b.at[1]], add=True)

            pltpu.make_async_copy(y_hbm.at[idx_a.at[0]], rows_a,
                                  sem_a).wait()
            pltpu.sync_copy(rows_a, acc.at[idx_a.at[1]], add=True)

        @pl.when(c == 0)
        def _():
            run_range(s * G_SLOW, G_SLOW)

        @pl.when(c != 0)
        def _():
            run_range(NS * G_SLOW + s * G_FAST, G_FAST)

        plsc.subcore_barrier()
        pltpu.sync_copy(
            acc.at[pl.ds(s * RPS, RPS)],
            out_hbm.at[c, pl.ds(s * RPS, RPS)],
        )

    return spmm_kernel(y, edge_chunks, zeros_s)


_RB = 1000
_GRID = N // _RB


def _tc1_body(degs_ref, x_ref, w1_ref, xw_ref, y1_ref, dinv_ref):
    deg = degs_ref[0, :, 0:1] + degs_ref[1, :, 0:1] + 1.0
    dinv = lax.rsqrt(deg)
    xw = jnp.dot(x_ref[...], w1_ref[...], preferred_element_type=jnp.float32)
    xw_ref[...] = xw
    y1_ref[...] = xw * dinv
    dinv_ref[...] = jnp.broadcast_to(dinv, (_RB, CH))


def _tc1(degs, x, W1):
    return pl.pallas_call(
        _tc1_body,
        grid=(_GRID,),
        in_specs=[
            pl.BlockSpec((NC, _RB, 16), lambda i: (0, i, 0)),
            pl.BlockSpec((_RB, CH), lambda i: (i, 0)),
            pl.BlockSpec((CH, CH), lambda i: (0, 0)),
        ],
        out_specs=[
            pl.BlockSpec((_RB, CH), lambda i: (i, 0)),
            pl.BlockSpec((_RB, CH), lambda i: (i, 0)),
            pl.BlockSpec((_RB, CH), lambda i: (i, 0)),
        ],
        out_shape=[
            jax.ShapeDtypeStruct((N, CH), jnp.float32),
            jax.ShapeDtypeStruct((N, CH), jnp.float32),
            jax.ShapeDtypeStruct((N, CH), jnp.float32),
        ],
    )(degs, x, W1)


def _tc2_body(p_ref, xw_ref, dinv_ref, b1_ref, h_ref, y2_ref):
    dinv = dinv_ref[...]
    z = dinv * (p_ref[0] + p_ref[1]) + dinv * dinv * xw_ref[...]
    h = jnp.maximum(z + b1_ref[...], 0.0)
    h_ref[...] = h
    y2_ref[...] = h * dinv


def _tc2(p1, xw, dinvb, b1):
    return pl.pallas_call(
        _tc2_body,
        grid=(_GRID,),
        in_specs=[
            pl.BlockSpec((NC, _RB, CH), lambda i: (0, i, 0)),
            pl.BlockSpec((_RB, CH), lambda i: (i, 0)),
            pl.BlockSpec((_RB, CH), lambda i: (i, 0)),
            pl.BlockSpec((1, CH), lambda i: (0, 0)),
        ],
        out_specs=[
            pl.BlockSpec((_RB, CH), lambda i: (i, 0)),
            pl.BlockSpec((_RB, CH), lambda i: (i, 0)),
        ],
        out_shape=[
            jax.ShapeDtypeStruct((N, CH), jnp.float32),
            jax.ShapeDtypeStruct((N, CH), jnp.float32),
        ],
    )(p1, xw, dinvb, b1)


def _tc3_body(p_ref, h_ref, dinv_ref, wmu_ref, bmu_ref, wls_ref, bls_ref,
              mu_ref, ls_ref):
    dinv = dinv_ref[...]
    z = dinv * (p_ref[0] + p_ref[1]) + dinv * dinv * h_ref[...]
    mu_ref[...] = (
        jnp.dot(z, wmu_ref[...], preferred_element_type=jnp.float32)
        + bmu_ref[...]
    )
    ls_ref[...] = (
        jnp.dot(z, wls_ref[...], preferred_element_type=jnp.float32)
        + bls_ref[...]
    )


def _tc3(p2, h, dinvb, W_mu, b_mu, W_ls, b_ls):
    return pl.pallas_call(
        _tc3_body,
        grid=(_GRID,),
        in_specs=[
            pl.BlockSpec((NC, _RB, CH), lambda i: (0, i, 0)),
            pl.BlockSpec((_RB, CH), lambda i: (i, 0)),
            pl.BlockSpec((_RB, CH), lambda i: (i, 0)),
            pl.BlockSpec((CH, OUT_CH), lambda i: (0, 0)),
            pl.BlockSpec((1, OUT_CH), lambda i: (0, 0)),
            pl.BlockSpec((CH, OUT_CH), lambda i: (0, 0)),
            pl.BlockSpec((1, OUT_CH), lambda i: (0, 0)),
        ],
        out_specs=[
            pl.BlockSpec((_RB, OUT_CH), lambda i: (i, 0)),
            pl.BlockSpec((_RB, OUT_CH), lambda i: (i, 0)),
        ],
        out_shape=[
            jax.ShapeDtypeStruct((N, OUT_CH), jnp.float32),
            jax.ShapeDtypeStruct((N, OUT_CH), jnp.float32),
        ],
    )(p2, h, dinvb, W_mu, b_mu, W_ls, b_ls)


def kernel(x, edge_index, W1, b1, W_mu, b_mu, W_ls, b_ls):
    src = edge_index[0].astype(jnp.int32)
    dst = edge_index[1].astype(jnp.int32)
    pad = E_PAD - E
    src_p = jnp.concatenate([src, jnp.zeros((pad,), jnp.int32)])
    dst_p = jnp.concatenate([dst, jnp.full((pad,), N, jnp.int32)])
    edge_chunks = jnp.stack(
        [src_p.reshape(NW * G, CHUNK), dst_p.reshape(NW * G, CHUNK)], axis=1
    ).reshape(NW * G, 2, CHUNK)

    ones_h = jnp.ones((CHUNK, 16), jnp.float32)
    zeros_h = jnp.zeros((RPS, 16), jnp.float32)
    zeros_s = jnp.zeros((RPS, CH), jnp.float32)

    degs = _sc_degree(edge_chunks, ones_h, zeros_h)
    xw, y1, dinvb = _tc1(degs, x, W1)
    p1 = _sc_spmm(y1, edge_chunks, zeros_s)
    h, y2 = _tc2(p1, xw, dinvb, b1.reshape(1, CH))
    p2 = _sc_spmm(y2, edge_chunks, zeros_s)
    mu, ls = _tc3(p2, h, dinvb, W_mu, b_mu.reshape(1, OUT_CH),
                  W_ls, b_ls.reshape(1, OUT_CH))
    return (mu, ls)


# speedup vs baseline: 1.3528x; 1.3528x over previous
import functools

import jax
import jax.numpy as jnp
from jax import lax
from jax.experimental import pallas as pl
from jax.experimental.pallas import tpu as pltpu
from jax.experimental.pallas import tpu_sc as plsc

N = 10000
E = 320000
CH = 128
OUT_CH = 64

NC = 2
NS = 16
NW = NC * NS
CHUNK = 128
G = -(-E // (NW * CHUNK))
EPW = G * CHUNK
E_PAD = EPW * NW
NCHUNKS = NW * G
G_C0 = 147
G_C1 = 2 * G - G_C0
RPS = 640
NROWS = RPS * NS

_sc_mesh = plsc.VectorSubcoreMesh(core_axis_name="c", subcore_axis_name="s")


def _sc_degree(edge_chunks, ones_h, zeros_h):
    @functools.partial(
        pl.kernel,
        out_type=jax.ShapeDtypeStruct((NC, NROWS, 16), jnp.float32),
        mesh=_sc_mesh,
        scratch_types=[
            pltpu.VMEM((2, CHUNK), jnp.int32),
            pltpu.VMEM((CHUNK, 16), jnp.float32),
            pltpu.VMEM_SHARED((NROWS, 16), jnp.float32),
        ],
    )
    def deg_kernel(edges_hbm, ones_hbm, zeros_hbm, out_hbm, idx_v, ones_v,
                   acc):
        c = lax.axis_index("c")
        s = lax.axis_index("s")
        wid = c * NS + s
        pltpu.sync_copy(ones_hbm, ones_v)
        pltpu.sync_copy(zeros_hbm, acc.at[pl.ds(s * RPS, RPS)])
        plsc.subcore_barrier()

        @pl.loop(0, G)
        def _(g):
            pltpu.sync_copy(edges_hbm.at[wid * G + g], idx_v)
            pltpu.sync_copy(ones_v, acc.at[idx_v.at[1]], add=True)

        plsc.subcore_barrier()
        pltpu.sync_copy(
            acc.at[pl.ds(s * RPS, RPS)],
            out_hbm.at[c, pl.ds(s * RPS, RPS)],
        )

    return deg_kernel(edge_chunks, ones_h, zeros_h)


def _sc_spmm(y, edge_chunks, zeros_s):
    @functools.partial(
        pl.kernel,
        out_type=jax.ShapeDtypeStruct((NC, NROWS, CH), jnp.float32),
        mesh=_sc_mesh,
        scratch_types=[
            pltpu.VMEM((2, CHUNK), jnp.int32),
            pltpu.VMEM((2, CHUNK), jnp.int32),
            pltpu.VMEM((CHUNK, CH), jnp.float32),
            pltpu.VMEM((CHUNK, CH), jnp.float32),
            pltpu.VMEM_SHARED((NROWS, CH), jnp.float32),
            pltpu.SemaphoreType.DMA,
            pltpu.SemaphoreType.DMA,
        ],
    )
    def spmm_kernel(y_hbm, edges_hbm, zeros_hbm, out_hbm,
                    idx_a, idx_b, rows_a, rows_b, acc, sem_a, sem_b):
        c = lax.axis_index("c")
        s = lax.axis_index("s")

        pltpu.sync_copy(zeros_hbm, acc.at[pl.ds(s * RPS, RPS)])
        plsc.subcore_barrier()

        def run_range(base0, cnt):
            pltpu.sync_copy(edges_hbm.at[base0], idx_a)
            pltpu.async_copy(y_hbm.at[idx_a.at[0]], rows_a, sem_a)

            @pl.loop(0, cnt - 2, step=2)
            def _(g):
                base = base0 + g
                pltpu.sync_copy(edges_hbm.at[base + 1], idx_b)
                pltpu.async_copy(y_hbm.at[idx_b.at[0]], rows_b, sem_b)
                pltpu.make_async_copy(y_hbm.at[idx_a.at[0]], rows_a,
                                      sem_a).wait()
                pltpu.sync_copy(rows_a, acc.at[idx_a.at[1]], add=True)
                pltpu.sync_copy(edges_hbm.at[base + 2], idx_a)
                pltpu.async_copy(y_hbm.at[idx_a.at[0]], rows_a, sem_a)
                pltpu.make_async_copy(y_hbm.at[idx_b.at[0]], rows_b,
                                      sem_b).wait()
                pltpu.sync_copy(rows_b, acc.at[idx_b.at[1]], add=True)

            pltpu.make_async_copy(y_hbm.at[idx_a.at[0]], rows_a,
                                  sem_a).wait()
            pltpu.sync_copy(rows_a, acc.at[idx_a.at[1]], add=True)

        @pl.when(c == 0)
        def _():
            run_range(s * G_C0, G_C0)

        @pl.when(c != 0)
        def _():
            run_range(NS * G_C0 + s * G_C1, G_C1)

        plsc.subcore_barrier()
        pltpu.sync_copy(
            acc.at[pl.ds(s * RPS, RPS)],
            out_hbm.at[c, pl.ds(s * RPS, RPS)],
        )

    return spmm_kernel(y, edge_chunks, zeros_s)


_RB = 1000
_GRID = N // _RB


def _tc1_body(degs_ref, x_ref, w1_ref, xw_ref, y1_ref, dinv_ref):
    deg = degs_ref[0, :, 0:1] + degs_ref[1, :, 0:1] + 1.0
    dinv = lax.rsqrt(deg)
    xw = jnp.dot(x_ref[...], w1_ref[...], preferred_element_type=jnp.float32)
    xw_ref[...] = xw
    y1_ref[...] = xw * dinv
    dinv_ref[...] = jnp.broadcast_to(dinv, (_RB, CH))


def _tc1(degs, x, W1):
    return pl.pallas_call(
        _tc1_body,
        grid=(_GRID,),
        in_specs=[
            pl.BlockSpec((NC, _RB, 16), lambda i: (0, i, 0)),
            pl.BlockSpec((_RB, CH), lambda i: (i, 0)),
            pl.BlockSpec((CH, CH), lambda i: (0, 0)),
        ],
        out_specs=[
            pl.BlockSpec((_RB, CH), lambda i: (i, 0)),
            pl.BlockSpec((_RB, CH), lambda i: (i, 0)),
            pl.BlockSpec((_RB, CH), lambda i: (i, 0)),
        ],
        out_shape=[
            jax.ShapeDtypeStruct((N, CH), jnp.float32),
            jax.ShapeDtypeStruct((N, CH), jnp.float32),
            jax.ShapeDtypeStruct((N, CH), jnp.float32),
        ],
    )(degs, x, W1)


def _tc2_body(p_ref, xw_ref, dinv_ref, b1_ref, h_ref, y2_ref):
    dinv = dinv_ref[...]
    z = dinv * (p_ref[0] + p_ref[1]) + dinv * dinv * xw_ref[...]
    h = jnp.maximum(z + b1_ref[...], 0.0)
    h_ref[...] = h
    y2_ref[...] = h * dinv


def _tc2(p1, xw, dinvb, b1):
    return pl.pallas_call(
        _tc2_body,
        grid=(_GRID,),
        in_specs=[
            pl.BlockSpec((NC, _RB, CH), lambda i: (0, i, 0)),
            pl.BlockSpec((_RB, CH), lambda i: (i, 0)),
            pl.BlockSpec((_RB, CH), lambda i: (i, 0)),
            pl.BlockSpec((1, CH), lambda i: (0, 0)),
        ],
        out_specs=[
            pl.BlockSpec((_RB, CH), lambda i: (i, 0)),
            pl.BlockSpec((_RB, CH), lambda i: (i, 0)),
        ],
        out_shape=[
            jax.ShapeDtypeStruct((N, CH), jnp.float32),
            jax.ShapeDtypeStruct((N, CH), jnp.float32),
        ],
    )(p1, xw, dinvb, b1)


def _tc3_body(p_ref, h_ref, dinv_ref, wmu_ref, bmu_ref, wls_ref, bls_ref,
              mu_ref, ls_ref):
    dinv = dinv_ref[...]
    z = dinv * (p_ref[0] + p_ref[1]) + dinv * dinv * h_ref[...]
    mu_ref[...] = (
        jnp.dot(z, wmu_ref[...], preferred_element_type=jnp.float32)
        + bmu_ref[...]
    )
    ls_ref[...] = (
        jnp.dot(z, wls_ref[...], preferred_element_type=jnp.float32)
        + bls_ref[...]
    )


def _tc3(p2, h, dinvb, W_mu, b_mu, W_ls, b_ls):
    return pl.pallas_call(
        _tc3_body,
        grid=(_GRID,),
        in_specs=[
            pl.BlockSpec((NC, _RB, CH), lambda i: (0, i, 0)),
            pl.BlockSpec((_RB, CH), lambda i: (i, 0)),
            pl.BlockSpec((_RB, CH), lambda i: (i, 0)),
            pl.BlockSpec((CH, OUT_CH), lambda i: (0, 0)),
            pl.BlockSpec((1, OUT_CH), lambda i: (0, 0)),
            pl.BlockSpec((CH, OUT_CH), lambda i: (0, 0)),
            pl.BlockSpec((1, OUT_CH), lambda i: (0, 0)),
        ],
        out_specs=[
            pl.BlockSpec((_RB, OUT_CH), lambda i: (i, 0)),
            pl.BlockSpec((_RB, OUT_CH), lambda i: (i, 0)),
        ],
        out_shape=[
            jax.ShapeDtypeStruct((N, OUT_CH), jnp.float32),
            jax.ShapeDtypeStruct((N, OUT_CH), jnp.float32),
        ],
    )(p2, h, dinvb, W_mu, b_mu, W_ls, b_ls)


def kernel(x, edge_index, W1, b1, W_mu, b_mu, W_ls, b_ls):
    src = edge_index[0].astype(jnp.int32)
    dst = edge_index[1].astype(jnp.int32)
    pad = E_PAD - E
    src_p = jnp.concatenate([src, jnp.zeros((pad,), jnp.int32)])
    dst_p = jnp.concatenate([dst, jnp.full((pad,), N, jnp.int32)])
    edge_chunks = jnp.stack(
        [src_p.reshape(NW * G, CHUNK), dst_p.reshape(NW * G, CHUNK)], axis=1
    ).reshape(NW * G, 2, CHUNK)

    ones_h = jnp.ones((CHUNK, 16), jnp.float32)
    zeros_h = jnp.zeros((RPS, 16), jnp.float32)
    zeros_s = jnp.zeros((RPS, CH), jnp.float32)

    degs = _sc_degree(edge_chunks, ones_h, zeros_h)
    xw, y1, dinvb = _tc1(degs, x, W1)
    p1 = _sc_spmm(y1, edge_chunks, zeros_s)
    h, y2 = _tc2(p1, xw, dinvb, b1.reshape(1, CH))
    p2 = _sc_spmm(y2, edge_chunks, zeros_s)
    mu, ls = _tc3(p2, h, dinvb, W_mu, b_mu.reshape(1, OUT_CH),
                  W_ls, b_ls.reshape(1, OUT_CH))
    return (mu, ls)
